# baseline (speedup 1.0000x reference)
import functools

import jax
import jax.numpy as jnp
from jax import lax
from jax.experimental import pallas as pl
from jax.experimental.pallas import tpu as pltpu

B, F, N, E, K = 128, 128, 4096, 64, 10
NEG = -1e30
BT = 8
PR, PC = 8, N // 8


def _dist_col(q_row, fmat):
    qq = jnp.sum(q_row * q_row)
    fsq = jnp.sum(fmat * fmat, axis=1, keepdims=True)
    qf = lax.dot_general(fmat, q_row, (((1,), (1,)), ((), ())),
                         preferred_element_type=jnp.float32)
    return jnp.sqrt(jnp.maximum(qq + fsq - 2.0 * qf, 0.0) + 1e-12)


def _hoppy_kernel(rel_r, a1_r, a2_r, frel_r, fa1_r, fa2_r, nbf_r, ent_r,
                  nbe_r, w1_r, w2_r, s0_r, res_r):
    flat_iota = (lax.broadcasted_iota(jnp.int32, (PR, PC), 0) * PC
                 + lax.broadcasted_iota(jnp.int32, (PR, PC), 1))

    for j in range(BT):
        b = pl.program_id(0) * BT + j
        relrow = rel_r[pl.ds(b, 1), :]
        a1row = a1_r[pl.ds(b, 1), :]
        a2row = a2_r[pl.ds(b, 1), :]
        frel = frel_r[j]
        fa1 = fa1_r[j]
        fa2 = fa2_r[j]
        ents = ent_r[j]
        nbf = nbf_r[b]
        nbe = nbe_r[b]

        h1 = jnp.dot(relrow, w1_r[...], preferred_element_type=jnp.float32)
        h2 = jnp.dot(relrow, w2_r[...], preferred_element_type=jnp.float32)

        fvalid = lax.broadcasted_iota(jnp.int32, (F, 1), 0) < nbf
        r0r = _dist_col(relrow, frel)
        r0a = _dist_col(a1row, fa1)
        r0b = _dist_col(a2row, fa2)
        ls0 = jnp.max(jnp.where(fvalid, -(r0r + r0a + r0b), NEG),
                      axis=0, keepdims=True)
        lw1 = jnp.where(fvalid, -(_dist_col(h1, frel) + r0a), NEG)
        lw2 = jnp.where(fvalid, -(_dist_col(h2, frel) + r0b), NEG)

        entsb = ents.astype(jnp.bfloat16)
        sqe = lax.dot_general(jnp.ones((1, E), jnp.bfloat16), entsb * entsb,
                              (((1,), (1,)), ((), ())),
                              preferred_element_type=jnp.float32
                              ).astype(jnp.bfloat16)
        def ent_row(fmat, lw):
            fmatb = (fmat * -2.0).astype(jnp.bfloat16)
            mm = lax.dot_general(fmatb, entsb, (((1,), (1,)), ((), ())),
                                 preferred_element_type=jnp.float32
                                 ).astype(jnp.bfloat16)
            fsq = jnp.sum(fmat * fmat, axis=1,
                          keepdims=True).astype(jnp.bfloat16)
            rr = jnp.sqrt(sqe + (fsq + mm))
            red = jnp.max(lw.astype(jnp.bfloat16) - rr, axis=0,
                          keepdims=True)
            return jnp.reshape(red, (PR, PC)).astype(jnp.float32)

        mp = ent_row(fa2, lw1)
        h2p = ent_row(fa1, lw2)
        mp = jnp.where(flat_iota < nbe, mp, NEG)

        cur = mp
        zv = None
        for _ in range(K):
            zv = jnp.max(cur, axis=(0, 1), keepdims=True)
            cur = jnp.where(cur == zv, NEG, cur)

        t = jnp.where(mp >= zv, jnp.minimum(mp, h2p), NEG)
        g = jnp.max(jnp.max(t, axis=1, keepdims=True), axis=0, keepdims=True)
        res = jnp.exp(jnp.maximum(ls0, g))
        s0_r[j] = jnp.broadcast_to(jnp.exp(ls0), (1, 128))
        res_r[j] = jnp.broadcast_to(res, (1, 128))


@functools.partial(jax.jit, static_argnames=("interpret",))
def _run(rel, arg1, arg2, fact_rel, fact_arg1, fact_arg2, nb_facts,
         entity_embeddings, nb_entities, W1, W2, interpret=False):
    row = lambda: pl.BlockSpec((B, E), lambda b: (0, 0))
    fac = lambda: pl.BlockSpec((BT, F, E), lambda b: (b, 0, 0))
    smem = lambda: pl.BlockSpec((B,), lambda b: (0,), memory_space=pltpu.SMEM)
    s0, res = pl.pallas_call(
        _hoppy_kernel,
        grid=(B // BT,),
        in_specs=[
            row(), row(), row(), fac(), fac(), fac(), smem(),
            pl.BlockSpec((BT, N, E), lambda b: (b, 0, 0)), smem(),
            pl.BlockSpec((E, E), lambda b: (0, 0)),
            pl.BlockSpec((E, E), lambda b: (0, 0)),
        ],
        out_specs=[pl.BlockSpec((BT, 1, 128), lambda b: (b, 0, 0))] * 2,
        out_shape=[jax.ShapeDtypeStruct((B, 1, 128), jnp.float32)] * 2,
        interpret=interpret,
    )(rel, arg1, arg2, fact_rel, fact_arg1, fact_arg2,
      nb_facts.astype(jnp.int32), entity_embeddings,
      nb_entities.astype(jnp.int32), W1, W2)
    return s0[:, 0, 0], res[:, 0, 0]


def kernel(rel, arg1, arg2, fact_rel, fact_arg1, fact_arg2, nb_facts,
           entity_embeddings, nb_entities, W1, W2, depth):
    s0, res = _run(rel, arg1, arg2, fact_rel, fact_arg1, fact_arg2,
                   nb_facts, entity_embeddings, nb_entities, W1, W2)
    return jnp.where(depth <= 0, s0, res)


# speedup vs baseline: 1.0089x; 1.0089x over previous
import functools

import jax
import jax.numpy as jnp
from jax import lax
from jax.experimental import pallas as pl
from jax.experimental.pallas import tpu as pltpu

B, F, N, E, K = 128, 128, 4096, 64, 10
NEG = -1e30
BT = 4
PR, PC = 8, N // 8


def _dist_col(q_row, fmat):
    qq = jnp.sum(q_row * q_row)
    fsq = jnp.sum(fmat * fmat, axis=1, keepdims=True)
    qf = lax.dot_general(fmat, q_row, (((1,), (1,)), ((), ())),
                         preferred_element_type=jnp.float32)
    return jnp.sqrt(jnp.maximum(qq + fsq - 2.0 * qf, 0.0) + 1e-12)


def _hoppy_kernel(rel_r, a1_r, a2_r, frel_r, fa1_r, fa2_r, nbf_r, ent_r,
                  nbe_r, w1_r, w2_r, s0_r, res_r):
    flat_iota = (lax.broadcasted_iota(jnp.int32, (PR, PC), 0) * PC
                 + lax.broadcasted_iota(jnp.int32, (PR, PC), 1))

    for j in range(BT):
        b = pl.program_id(0) * BT + j
        relrow = rel_r[pl.ds(b, 1), :]
        a1row = a1_r[pl.ds(b, 1), :]
        a2row = a2_r[pl.ds(b, 1), :]
        frel = frel_r[j]
        fa1 = fa1_r[j]
        fa2 = fa2_r[j]
        ents = ent_r[j]
        nbf = nbf_r[b]
        nbe = nbe_r[b]

        h1 = jnp.dot(relrow, w1_r[...], preferred_element_type=jnp.float32)
        h2 = jnp.dot(relrow, w2_r[...], preferred_element_type=jnp.float32)

        fvalid = lax.broadcasted_iota(jnp.int32, (F, 1), 0) < nbf
        r0r = _dist_col(relrow, frel)
        r0a = _dist_col(a1row, fa1)
        r0b = _dist_col(a2row, fa2)
        ls0 = jnp.max(jnp.where(fvalid, -(r0r + r0a + r0b), NEG),
                      axis=0, keepdims=True)
        lw1 = jnp.where(fvalid, -(_dist_col(h1, frel) + r0a), NEG)
        lw2 = jnp.where(fvalid, -(_dist_col(h2, frel) + r0b), NEG)

        entsb = ents.astype(jnp.bfloat16)
        sqe = lax.dot_general(jnp.ones((1, E), jnp.bfloat16), entsb * entsb,
                              (((1,), (1,)), ((), ())),
                              preferred_element_type=jnp.float32
                              ).astype(jnp.bfloat16)
        def ent_row(fmat, lw):
            fmatb = (fmat * -2.0).astype(jnp.bfloat16)
            mm = lax.dot_general(fmatb, entsb, (((1,), (1,)), ((), ())),
                                 preferred_element_type=jnp.float32
                                 ).astype(jnp.bfloat16)
            fsq = jnp.sum(fmat * fmat, axis=1,
                          keepdims=True).astype(jnp.bfloat16)
            rr = jnp.sqrt(sqe + (fsq + mm))
            red = jnp.max(lw.astype(jnp.bfloat16) - rr, axis=0,
                          keepdims=True)
            return jnp.reshape(red, (PR, PC)).astype(jnp.float32)

        mp = ent_row(fa2, lw1)
        h2p = ent_row(fa1, lw2)
        mp = jnp.where(flat_iota < nbe, mp, NEG)

        cur = mp
        zv = None
        for _ in range(K):
            zv = jnp.max(cur, axis=(0, 1), keepdims=True)
            cur = jnp.where(cur == zv, NEG, cur)

        t = jnp.where(mp >= zv, jnp.minimum(mp, h2p), NEG)
        g = jnp.max(jnp.max(t, axis=1, keepdims=True), axis=0, keepdims=True)
        res = jnp.exp(jnp.maximum(ls0, g))
        s0_r[j] = jnp.broadcast_to(jnp.exp(ls0), (1, 128))
        res_r[j] = jnp.broadcast_to(res, (1, 128))


@functools.partial(jax.jit, static_argnames=("interpret",))
def _run(rel, arg1, arg2, fact_rel, fact_arg1, fact_arg2, nb_facts,
         entity_embeddings, nb_entities, W1, W2, interpret=False):
    row = lambda: pl.BlockSpec((B, E), lambda b: (0, 0))
    fac = lambda: pl.BlockSpec((BT, F, E), lambda b: (b, 0, 0))
    smem = lambda: pl.BlockSpec((B,), lambda b: (0,), memory_space=pltpu.SMEM)
    s0, res = pl.pallas_call(
        _hoppy_kernel,
        grid=(B // BT,),
        in_specs=[
            row(), row(), row(), fac(), fac(), fac(), smem(),
            pl.BlockSpec((BT, N, E), lambda b: (b, 0, 0)), smem(),
            pl.BlockSpec((E, E), lambda b: (0, 0)),
            pl.BlockSpec((E, E), lambda b: (0, 0)),
        ],
        out_specs=[pl.BlockSpec((BT, 1, 128), lambda b: (b, 0, 0))] * 2,
        out_shape=[jax.ShapeDtypeStruct((B, 1, 128), jnp.float32)] * 2,
        interpret=interpret,
    )(rel, arg1, arg2, fact_rel, fact_arg1, fact_arg2,
      nb_facts.astype(jnp.int32), entity_embeddings,
      nb_entities.astype(jnp.int32), W1, W2)
    return s0[:, 0, 0], res[:, 0, 0]


def kernel(rel, arg1, arg2, fact_rel, fact_arg1, fact_arg2, nb_facts,
           entity_embeddings, nb_entities, W1, W2, depth):
    s0, res = _run(rel, arg1, arg2, fact_rel, fact_arg1, fact_arg2,
                   nb_facts, entity_embeddings, nb_entities, W1, W2)
    return jnp.where(depth <= 0, s0, res)
